# Initial kernel scaffold; baseline (speedup 1.0000x reference)
import jax
import jax.numpy as jnp
from jax.experimental import pallas as pl


def kernel(history_tokens, history_post_tokens, history_author_tokens, history_action_tokens, history_time_gap, history_group_ids, lengths, token_table, time_table, group_table, pos_table, sep_token, ln_g, ln_b, W1, b1, W2, b2):
    raise NotImplementedError("write your pallas kernel here")


# speedup vs baseline: 3.5756x; 3.5756x over previous
import functools

import jax
import jax.numpy as jnp
from jax import lax
from jax.experimental import pallas as pl
from jax.experimental.pallas import tpu as pltpu
from jax.experimental.pallas import tpu_sc as plsc

B, L, T, H = 8, 256, 512, 128

_NC, _NS = 2, 16
_NW = _NC * _NS
_TOK_PER_W = 4 * B * L // _NW
_AUX_PER_W = B * L // _NW


def _sc_gather(tok_idx, time_idx, grp_idx, token_table, time_table, group_table):
  mesh = plsc.VectorSubcoreMesh(
      core_axis_name="c", subcore_axis_name="s",
      num_cores=_NC, num_subcores=_NS)

  @functools.partial(
      pl.kernel,
      out_type=(
          jax.ShapeDtypeStruct((4 * B * L, H), jnp.float32),
          jax.ShapeDtypeStruct((B * L, H), jnp.float32),
          jax.ShapeDtypeStruct((B * L, H), jnp.float32),
      ),
      mesh=mesh,
      scratch_types=(
          pltpu.VMEM((128,), jnp.int32),
          pltpu.VMEM((128, H), jnp.float32),
          pltpu.VMEM((_AUX_PER_W,), jnp.int32),
          pltpu.VMEM((_AUX_PER_W, H), jnp.float32),
          pltpu.SemaphoreType.DMA,
      ),
  )
  def gather_kernel(tok_idx_hbm, time_idx_hbm, grp_idx_hbm,
                    tok_tab, time_tab, grp_tab,
                    tok_out, time_out, grp_out,
                    idx_v, rows_v, idx_s, rows_s, sem):
    wid = lax.axis_index("s") * _NC + lax.axis_index("c")
    for j in range(_TOK_PER_W // 128):
      base = wid * _TOK_PER_W + j * 128
      pltpu.sync_copy(tok_idx_hbm.at[pl.ds(base, 128)], idx_v)
      pltpu.async_copy(tok_tab.at[idx_v], rows_v, sem).wait()
      pltpu.sync_copy(rows_v, tok_out.at[pl.ds(base, 128)])
    abase = wid * _AUX_PER_W
    pltpu.sync_copy(time_idx_hbm.at[pl.ds(abase, _AUX_PER_W)], idx_s)
    pltpu.async_copy(time_tab.at[idx_s], rows_s, sem).wait()
    pltpu.sync_copy(rows_s, time_out.at[pl.ds(abase, _AUX_PER_W)])
    pltpu.sync_copy(grp_idx_hbm.at[pl.ds(abase, _AUX_PER_W)], idx_s)
    pltpu.async_copy(grp_tab.at[idx_s], rows_s, sem).wait()
    pltpu.sync_copy(rows_s, grp_out.at[pl.ds(abase, _AUX_PER_W)])

  return gather_kernel(tok_idx, time_idx, grp_idx,
                       token_table, time_table, group_table)


def _tc_body(tok_ref, time_ref, grp_ref, gid_ref, len_ref,
             pos_ref, sep_ref, g_ref, bln_ref, w1_ref, b1_ref, w2_ref, b2_ref,
             out_ref, mask_ref):
  f32 = jnp.float32
  b = pl.program_id(0)

  x = jnp.concatenate(
      [tok_ref[0, 0], tok_ref[1, 0], tok_ref[2, 0], tok_ref[3, 0],
       time_ref[0], grp_ref[0]], axis=-1)
  mu = jnp.mean(x, axis=-1, keepdims=True)
  xc = x - mu
  var = jnp.mean(xc * xc, axis=-1, keepdims=True)
  xn = xc * lax.rsqrt(var + 1e-5) * g_ref[...] + bln_ref[...]
  h = jnp.dot(xn, w1_ref[...], preferred_element_type=f32) + b1_ref[...]
  h = h * (1.0 / (1.0 + jnp.exp(-h)))
  ev = jnp.dot(h, w2_ref[...], preferred_element_type=f32) + b2_ref[...]

  n = len_ref[b]
  gid = gid_ref[0]
  idx = lax.broadcasted_iota(jnp.int32, (1, L), 1)
  g_next = jnp.concatenate([gid[:, 1:], gid[:, -1:]], axis=1)
  sep = (idx + 1 < n) & (gid != g_next)
  sep_f = sep.astype(f32)
  ii = lax.broadcasted_iota(jnp.int32, (L, L), 0)
  jj = lax.broadcasted_iota(jnp.int32, (L, L), 1)
  m_le = (ii <= jj).astype(f32)
  cum = jnp.dot(sep_f, m_le, preferred_element_type=f32)
  sep_before = (cum - sep_f).astype(jnp.int32)
  total_sep = jnp.max(cum).astype(jnp.int32)
  len_pieces = n + total_sep
  dest_tok = (T - len_pieces) + idx + sep_before
  tok_ok = (idx < n) & (dest_tok >= 0)
  sep_ok = sep & (dest_tok + 1 >= 0)
  dt = jnp.where(tok_ok, dest_tok, T)
  ds = jnp.where(sep_ok, dest_tok + 1, T)

  t_iota = lax.broadcasted_iota(jnp.int32, (T, L), 0)
  m_tok = (t_iota == dt).astype(f32)
  m_sep = (t_iota == ds).astype(f32)
  gathered = jnp.dot(m_tok, ev, preferred_element_type=f32)
  tok_hit = jnp.max(m_tok, axis=1, keepdims=True)
  sep_hit = jnp.max(m_sep, axis=1, keepdims=True)
  validf = jnp.maximum(tok_hit, sep_hit)
  merged = jnp.where(sep_hit > 0.0, sep_ref[...], gathered)
  out_ref[0] = (merged + pos_ref[...]) * validf
  mask_ref[0] = validf


def _tc_compute(tok4, time8, grp8, gids, lengths, pos_table, sep_row,
                ln_g, ln_b, W1, b1, W2, b2):
  grid = (B,)
  in_specs = [
      pl.BlockSpec((4, 1, L, H), lambda b: (0, b, 0, 0)),
      pl.BlockSpec((1, L, H), lambda b: (b, 0, 0)),
      pl.BlockSpec((1, L, H), lambda b: (b, 0, 0)),
      pl.BlockSpec((1, 1, L), lambda b: (b, 0, 0)),
      pl.BlockSpec(memory_space=pltpu.SMEM),
      pl.BlockSpec((T, H), lambda b: (0, 0)),
      pl.BlockSpec((1, H), lambda b: (0, 0)),
      pl.BlockSpec((1, 6 * H), lambda b: (0, 0)),
      pl.BlockSpec((1, 6 * H), lambda b: (0, 0)),
      pl.BlockSpec((6 * H, 4 * H), lambda b: (0, 0)),
      pl.BlockSpec((1, 4 * H), lambda b: (0, 0)),
      pl.BlockSpec((4 * H, H), lambda b: (0, 0)),
      pl.BlockSpec((1, H), lambda b: (0, 0)),
  ]
  out_specs = [
      pl.BlockSpec((1, T, H), lambda b: (b, 0, 0)),
      pl.BlockSpec((1, T, 1), lambda b: (b, 0, 0)),
  ]
  out_shape = [
      jax.ShapeDtypeStruct((B, T, H), jnp.float32),
      jax.ShapeDtypeStruct((B, T, 1), jnp.float32),
  ]
  return pl.pallas_call(
      _tc_body, grid=grid, in_specs=in_specs, out_specs=out_specs,
      out_shape=out_shape,
  )(tok4, time8, grp8, gids, lengths, pos_table, sep_row,
    ln_g, ln_b, W1, b1, W2, b2)


def kernel(history_tokens, history_post_tokens, history_author_tokens,
           history_action_tokens, history_time_gap, history_group_ids,
           lengths, token_table, time_table, group_table, pos_table,
           sep_token, ln_g, ln_b, W1, b1, W2, b2):
  tok_idx = jnp.concatenate([
      history_tokens.reshape(-1), history_post_tokens.reshape(-1),
      history_author_tokens.reshape(-1), history_action_tokens.reshape(-1),
  ]).astype(jnp.int32)
  time_idx = jnp.clip(history_time_gap, 0, 128).reshape(-1).astype(jnp.int32)
  grp_idx = history_group_ids.reshape(-1).astype(jnp.int32)

  tok_rows, time_rows, grp_rows = _sc_gather(
      tok_idx, time_idx, grp_idx, token_table, time_table, group_table)

  tok4 = tok_rows.reshape(4, B, L, H)
  time8 = time_rows.reshape(B, L, H)
  grp8 = grp_rows.reshape(B, L, H)
  gids = history_group_ids.astype(jnp.int32).reshape(B, 1, L)

  merged, maskf = _tc_compute(
      tok4, time8, grp8, gids, lengths.astype(jnp.int32), pos_table,
      sep_token.reshape(1, H), ln_g.reshape(1, 6 * H), ln_b.reshape(1, 6 * H),
      W1, b1.reshape(1, 4 * H), W2, b2.reshape(1, H))
  return merged, maskf.reshape(B, T) > 0.5


# speedup vs baseline: 3.8089x; 1.0653x over previous
import functools

import jax
import jax.numpy as jnp
from jax import lax
from jax.experimental import pallas as pl
from jax.experimental.pallas import tpu as pltpu
from jax.experimental.pallas import tpu_sc as plsc

B, L, T, H = 8, 256, 512, 128

_NC, _NS = 2, 16
_NW = _NC * _NS
_TOK_PER_W = 4 * B * L // _NW
_AUX_PER_W = B * L // _NW


def _sc_gather(tok_idx, time_idx, grp_idx, token_table, time_table, group_table):
  mesh = plsc.VectorSubcoreMesh(
      core_axis_name="c", subcore_axis_name="s",
      num_cores=_NC, num_subcores=_NS)

  @functools.partial(
      pl.kernel,
      out_type=(
          jax.ShapeDtypeStruct((4 * B * L, H), jnp.float32),
          jax.ShapeDtypeStruct((B * L, H), jnp.float32),
          jax.ShapeDtypeStruct((B * L, H), jnp.float32),
      ),
      mesh=mesh,
      scratch_types=(
          pltpu.VMEM((128,), jnp.int32),
          pltpu.VMEM((128,), jnp.int32),
          pltpu.VMEM((_AUX_PER_W,), jnp.int32),
          pltpu.VMEM((_AUX_PER_W,), jnp.int32),
          pltpu.VMEM((128, H), jnp.float32),
          pltpu.VMEM((128, H), jnp.float32),
          pltpu.VMEM((_AUX_PER_W, H), jnp.float32),
          pltpu.VMEM((_AUX_PER_W, H), jnp.float32),
          pltpu.SemaphoreType.DMA,
          pltpu.SemaphoreType.DMA,
          pltpu.SemaphoreType.DMA,
      ),
  )
  def gather_kernel(tok_idx_hbm, time_idx_hbm, grp_idx_hbm,
                    tok_tab, time_tab, grp_tab,
                    tok_out, time_out, grp_out,
                    idx_a, idx_b, idx_t, idx_g,
                    rows_a, rows_b, rows_t, rows_g,
                    sem_i, sem_g, sem_o):
    wid = lax.axis_index("s") * _NC + lax.axis_index("c")
    tb0 = wid * _TOK_PER_W
    tb1 = tb0 + 128
    ab = wid * _AUX_PER_W
    ci0 = pltpu.async_copy(tok_idx_hbm.at[pl.ds(tb0, 128)], idx_a, sem_i)
    ci1 = pltpu.async_copy(tok_idx_hbm.at[pl.ds(tb1, 128)], idx_b, sem_i)
    ci2 = pltpu.async_copy(time_idx_hbm.at[pl.ds(ab, _AUX_PER_W)], idx_t, sem_i)
    ci3 = pltpu.async_copy(grp_idx_hbm.at[pl.ds(ab, _AUX_PER_W)], idx_g, sem_i)
    ci0.wait(); ci1.wait(); ci2.wait(); ci3.wait()
    cg0 = pltpu.async_copy(tok_tab.at[idx_a], rows_a, sem_g)
    cg1 = pltpu.async_copy(tok_tab.at[idx_b], rows_b, sem_g)
    cg2 = pltpu.async_copy(time_tab.at[idx_t], rows_t, sem_g)
    cg3 = pltpu.async_copy(grp_tab.at[idx_g], rows_g, sem_g)
    cg0.wait(); cg1.wait(); cg2.wait(); cg3.wait()
    co0 = pltpu.async_copy(rows_a, tok_out.at[pl.ds(tb0, 128)], sem_o)
    co1 = pltpu.async_copy(rows_b, tok_out.at[pl.ds(tb1, 128)], sem_o)
    co2 = pltpu.async_copy(rows_t, time_out.at[pl.ds(ab, _AUX_PER_W)], sem_o)
    co3 = pltpu.async_copy(rows_g, grp_out.at[pl.ds(ab, _AUX_PER_W)], sem_o)
    co0.wait(); co1.wait(); co2.wait(); co3.wait()

  return gather_kernel(tok_idx, time_idx, grp_idx,
                       token_table, time_table, group_table)


def _tc_body(tok_ref, time_ref, grp_ref, gid_ref, len_ref,
             pos_ref, sep_ref, g_ref, bln_ref, w1_ref, b1_ref, w2_ref, b2_ref,
             out_ref, mask_ref):
  f32 = jnp.float32
  b = pl.program_id(0)

  x = jnp.concatenate(
      [tok_ref[0, 0], tok_ref[1, 0], tok_ref[2, 0], tok_ref[3, 0],
       time_ref[0], grp_ref[0]], axis=-1)
  mu = jnp.mean(x, axis=-1, keepdims=True)
  xc = x - mu
  var = jnp.mean(xc * xc, axis=-1, keepdims=True)
  xn = xc * lax.rsqrt(var + 1e-5) * g_ref[...] + bln_ref[...]
  h = jnp.dot(xn, w1_ref[...], preferred_element_type=f32) + b1_ref[...]
  h = h * (1.0 / (1.0 + jnp.exp(-h)))
  ev = jnp.dot(h, w2_ref[...], preferred_element_type=f32) + b2_ref[...]

  n = len_ref[b]
  gid = gid_ref[0]
  idx = lax.broadcasted_iota(jnp.int32, (1, L), 1)
  g_next = jnp.concatenate([gid[:, 1:], gid[:, -1:]], axis=1)
  sep = (idx + 1 < n) & (gid != g_next)
  sep_f = sep.astype(f32)
  ii = lax.broadcasted_iota(jnp.int32, (L, L), 0)
  jj = lax.broadcasted_iota(jnp.int32, (L, L), 1)
  m_le = (ii <= jj).astype(f32)
  cum = jnp.dot(sep_f, m_le, preferred_element_type=f32)
  sep_before = (cum - sep_f).astype(jnp.int32)
  total_sep = jnp.max(cum).astype(jnp.int32)
  len_pieces = n + total_sep
  dest_tok = (T - len_pieces) + idx + sep_before
  tok_ok = (idx < n) & (dest_tok >= 0)
  sep_ok = sep & (dest_tok + 1 >= 0)
  dt = jnp.where(tok_ok, dest_tok, T)
  ds = jnp.where(sep_ok, dest_tok + 1, T)

  t_iota = lax.broadcasted_iota(jnp.int32, (T, L), 0)
  m_tok = (t_iota == dt).astype(f32)
  m_sep = (t_iota == ds).astype(f32)
  gathered = jnp.dot(m_tok, ev, preferred_element_type=f32)
  tok_hit = jnp.max(m_tok, axis=1, keepdims=True)
  sep_hit = jnp.max(m_sep, axis=1, keepdims=True)
  validf = jnp.maximum(tok_hit, sep_hit)
  merged = jnp.where(sep_hit > 0.0, sep_ref[...], gathered)
  out_ref[0] = (merged + pos_ref[...]) * validf
  mask_ref[0] = validf


def _tc_compute(tok4, time8, grp8, gids, lengths, pos_table, sep_row,
                ln_g, ln_b, W1, b1, W2, b2):
  grid = (B,)
  in_specs = [
      pl.BlockSpec((4, 1, L, H), lambda b: (0, b, 0, 0)),
      pl.BlockSpec((1, L, H), lambda b: (b, 0, 0)),
      pl.BlockSpec((1, L, H), lambda b: (b, 0, 0)),
      pl.BlockSpec((1, 1, L), lambda b: (b, 0, 0)),
      pl.BlockSpec(memory_space=pltpu.SMEM),
      pl.BlockSpec((T, H), lambda b: (0, 0)),
      pl.BlockSpec((1, H), lambda b: (0, 0)),
      pl.BlockSpec((1, 6 * H), lambda b: (0, 0)),
      pl.BlockSpec((1, 6 * H), lambda b: (0, 0)),
      pl.BlockSpec((6 * H, 4 * H), lambda b: (0, 0)),
      pl.BlockSpec((1, 4 * H), lambda b: (0, 0)),
      pl.BlockSpec((4 * H, H), lambda b: (0, 0)),
      pl.BlockSpec((1, H), lambda b: (0, 0)),
  ]
  out_specs = [
      pl.BlockSpec((1, T, H), lambda b: (b, 0, 0)),
      pl.BlockSpec((1, T, 1), lambda b: (b, 0, 0)),
  ]
  out_shape = [
      jax.ShapeDtypeStruct((B, T, H), jnp.float32),
      jax.ShapeDtypeStruct((B, T, 1), jnp.float32),
  ]
  return pl.pallas_call(
      _tc_body, grid=grid, in_specs=in_specs, out_specs=out_specs,
      out_shape=out_shape,
  )(tok4, time8, grp8, gids, lengths, pos_table, sep_row,
    ln_g, ln_b, W1, b1, W2, b2)


def kernel(history_tokens, history_post_tokens, history_author_tokens,
           history_action_tokens, history_time_gap, history_group_ids,
           lengths, token_table, time_table, group_table, pos_table,
           sep_token, ln_g, ln_b, W1, b1, W2, b2):
  tok_idx = jnp.concatenate([
      history_tokens.reshape(-1), history_post_tokens.reshape(-1),
      history_author_tokens.reshape(-1), history_action_tokens.reshape(-1),
  ]).astype(jnp.int32)
  time_idx = jnp.clip(history_time_gap, 0, 128).reshape(-1).astype(jnp.int32)
  grp_idx = history_group_ids.reshape(-1).astype(jnp.int32)

  tok_rows, time_rows, grp_rows = _sc_gather(
      tok_idx, time_idx, grp_idx, token_table, time_table, group_table)

  tok4 = tok_rows.reshape(4, B, L, H)
  time8 = time_rows.reshape(B, L, H)
  grp8 = grp_rows.reshape(B, L, H)
  gids = history_group_ids.astype(jnp.int32).reshape(B, 1, L)

  merged, maskf = _tc_compute(
      tok4, time8, grp8, gids, lengths.astype(jnp.int32), pos_table,
      sep_token.reshape(1, H), ln_g.reshape(1, 6 * H), ln_b.reshape(1, 6 * H),
      W1, b1.reshape(1, 4 * H), W2, b2.reshape(1, H))
  return merged, maskf.reshape(B, T) > 0.5
